# baseline (speedup 1.0000x reference)
import jax
import jax.numpy as jnp
from jax.experimental import pallas as pl
from jax.experimental.pallas import tpu as pltpu

VMEM_LIMIT = 56 * 1024 * 1024

_BLOCKS = [
    (24,  32, False, True),
    (32,  48, False, True),
    (48,  64, True,  True),
    (64,  92, False, True),
    (76, 108, True,  True),
    (76, 108, False, True),
    (76, 108, True,  True),
    (76, 108, False, True),
    (76, 108, True,  False),
]

_STAGES = [
    (True, [0, 1, 2, 3, 4, 5, 6, 7, 8], True),
]

_B = 8


def _names(with_k5):
    return ["s2_2", "s1_0", "s2_0", "s1_2"] + (["s1_1", "s2_1"] if with_k5 else [])


def _cout(c1, c2, with_k5):
    return (3 * c2 + c1) if with_k5 else (2 * c2 + c1)


def _prelu(y, a):
    return jnp.where(y >= 0.0, y, y * a)


def _conv1x1(x2, w, b, a):
    y = jnp.dot(x2, w[...], preferred_element_type=jnp.float32)
    return _prelu(y + b[...], a[...])


def _shift_pool(x4):
    bb, h, w, c = x4.shape
    zc = jnp.zeros((bb, h, 1, c), x4.dtype)
    xw = jnp.concatenate([x4, zc], axis=2)
    zr = jnp.zeros((bb, 1, w + 1, c), x4.dtype)
    xp = jnp.concatenate([xw, zr], axis=1)
    return (xp[:, :h, :w] + xp[:, 1:, :w] + xp[:, :h, 1:] + xp[:, 1:, 1:]) * 0.25


def _pool2(x4):
    bb, h, w, c = x4.shape
    r = x4.reshape(bb, h // 2, 2, w, c)
    rs = r[:, :, 0] + r[:, :, 1]
    s = rs.reshape(bb, h // 2, w // 2, 2, c)
    return (s[:, :, :, 0] + s[:, :, :, 1]) * 0.25


def _convkxk(x2, k, h, w, c1, wref, bref, aref, pad_ref):
    p = (k - 1) // 2
    bb = pad_ref.shape[0]
    pad_ref[...] = jnp.zeros_like(pad_ref)
    pad_ref[:, p:p + h, p:p + w, :] = x2.reshape(bb, h, w, c1).astype(pad_ref.dtype)
    acc = None
    for dy in range(k):
        cols = [pad_ref[:, dy:dy + h, dx:dx + w, :].reshape(bb * h * w, c1)
                for dx in range(k)]
        im = jnp.concatenate(cols, axis=1)
        part = jnp.dot(im, wref[dy * k * c1:(dy + 1) * k * c1, :],
                       preferred_element_type=jnp.float32)
        acc = part if acc is None else acc + part
    return _prelu(acc + bref[...], aref[...])


def _make_stage_kernel(has_conv0, bidx, has_fc):
    cfgs = [_BLOCKS[i] for i in bidx]

    def body(*refs):
        it = iter(refs)
        x_ref = next(it)
        if has_conv0:
            cw, cb, ca = next(it), next(it), next(it)
        prms = []
        for (c1, c2, pool_in, with_k5) in cfgs:
            prms.append({nm: (next(it), next(it), next(it))
                         for nm in _names(with_k5)})
        if has_fc:
            fw, fb = next(it), next(it)
        o_ref = next(it)
        if has_conv0:
            pad0_ref = next(it)
        pads = []
        for (c1, c2, pool_in, with_k5) in cfgs:
            p5 = next(it) if with_k5 else None
            pads.append((p5, next(it)))
        n_act = len(cfgs) if has_fc else len(cfgs) - 1
        acts = [next(it) for _ in range(n_act)]
        bb = _B

        if has_conv0:
            h = w = 16
            y = _convkxk(x_ref[...].reshape(bb * h * w, 16), 3, h, w, 16,
                         cw, cb, ca, pad0_ref)
            prev = (0.25 * (y[:, 0:64] + y[:, 64:128] +
                            y[:, 128:192] + y[:, 192:256])
                    ).reshape(bb, h, w, 64)
        else:
            prev = x_ref[...]
            h = w = prev.shape[1]

        for bi, ((c1, c2, pool_in, with_k5), prm) in enumerate(zip(cfgs, prms)):
            if pool_in:
                prev = _pool2(prev)
                h //= 2
                w //= 2
            cin = prev.shape[-1]
            x2 = prev.reshape(bb * h * w, cin).astype(jnp.bfloat16)
            p5_ref, p3_ref = pads[bi]
            out_ref = acts[bi] if bi < len(acts) else o_ref

            s22 = _conv1x1(x2, *prm["s2_2"])
            out_ref[:, :, :, 0:c2] = s22.reshape(bb, h, w, c2)
            off = c2
            if with_k5:
                s11 = _conv1x1(x2, *prm["s1_1"])
                y21 = _convkxk(s11, 5, h, w, c1, *prm["s2_1"], p5_ref)
                out_ref[:, :, :, off:off + c2] = y21.reshape(bb, h, w, c2)
                off += c2
            s10 = _conv1x1(x2, *prm["s1_0"])
            y20 = _convkxk(s10, 3, h, w, c1, *prm["s2_0"], p3_ref)
            out_ref[:, :, :, off:off + c2] = y20.reshape(bb, h, w, c2)
            off += c2
            s12 = _conv1x1(x2, *prm["s1_2"])
            out_ref[:, :, :, off:off + c1] = _shift_pool(s12.reshape(bb, h, w, c1))

            prev = out_ref[...]

        if has_fc:
            flat = prev.reshape(bb, prev.shape[-1]).astype(jnp.bfloat16)
            o_ref[...] = (jnp.dot(flat, fw[...],
                                  preferred_element_type=jnp.float32) + fb[...])

    return body


def _zero_map(rank):
    def index_map(n):
        return (0,) * rank
    return index_map


def _conv0_s2d_weight(w):
    w2 = jnp.zeros((3, 3, 16, 256), jnp.float32)
    for ty in range(2):
        for tx in range(2):
            for dy5 in range(5):
                for dx5 in range(5):
                    by, sy = divmod(ty - 2 + dy5, 2)
                    bx, sx = divmod(tx - 2 + dx5, 2)
                    ci = (sy * 2 + sx) * 4
                    co = (ty * 2 + tx) * 64
                    w2 = w2.at[by + 1, bx + 1, ci:ci + 4,
                               co:co + 64].set(w[dy5, dx5])
    return w2.reshape(9 * 16, 256)


def kernel(x, conv0_w, conv0_b, conv0_a, *rest):
    N = x.shape[0]
    fc0_w, fc0_b = rest[-2], rest[-1]
    rest = rest[:-2]

    blk_prm = []
    idx = 0
    for (c1, c2, pool_in, with_k5) in _BLOCKS:
        n_sub = 6 if with_k5 else 4
        sub = rest[idx:idx + 3 * n_sub]
        idx += 3 * n_sub
        w22, b22, a22, w10, b10, a10, w20, b20, a20, w12, b12, a12 = sub[:12]
        bf = jnp.bfloat16
        prm = {"s2_2": (w22.astype(bf), b22, a22),
               "s1_0": (w10.astype(bf), b10, a10),
               "s2_0": (w20.reshape(9 * c1, c2).astype(bf), b20, a20),
               "s1_2": (w12.astype(bf), b12, a12)}
        if with_k5:
            w11, b11, a11, w21, b21, a21 = sub[12:]
            prm["s1_1"] = (w11.astype(bf), b11, a11)
            prm["s2_1"] = (w21.reshape(25 * c1, c2).astype(bf), b21, a21)
        blk_prm.append(prm)

    cur = x.reshape(N, 4, 16, 2, 16, 2).transpose(0, 2, 4, 3, 5, 1)
    cur = cur.reshape(N, 16, 16, 16)
    cw2 = _conv0_s2d_weight(conv0_w).astype(jnp.bfloat16)
    cb2 = jnp.tile(conv0_b, (1, 4))
    ca2 = jnp.tile(conv0_a, (1, 4))

    h = 16
    for (has_conv0, bidx, has_fc) in _STAGES:
        operands = [cur]
        in_specs = [pl.BlockSpec((_B,) + cur.shape[1:],
                                 lambda n: (n, 0, 0, 0))]

        def add(arr):
            operands.append(arr)
            in_specs.append(pl.BlockSpec(arr.shape, _zero_map(arr.ndim)))

        if has_conv0:
            add(cw2)
            add(cb2)
            add(ca2)
        for i in bidx:
            c1, c2, pool_in, with_k5 = _BLOCKS[i]
            for nm in _names(with_k5):
                w, b, a = blk_prm[i][nm]
                add(w); add(b); add(a)
        if has_fc:
            add(fc0_w.astype(jnp.bfloat16))
            add(fc0_b)

        scratch = []
        if has_conv0:
            scratch.append(pltpu.VMEM((_B, 18, 18, 16), jnp.bfloat16))
        hh = h
        for i in bidx:
            c1, c2, pool_in, with_k5 = _BLOCKS[i]
            if pool_in:
                hh //= 2
            if with_k5:
                scratch.append(pltpu.VMEM((_B, hh + 4, hh + 4, c1), jnp.bfloat16))
            scratch.append(pltpu.VMEM((_B, hh + 2, hh + 2, c1), jnp.bfloat16))
        hh = h
        act_shapes = []
        for i in bidx:
            c1, c2, pool_in, with_k5 = _BLOCKS[i]
            if pool_in:
                hh //= 2
            act_shapes.append((_B, hh, hh, _cout(c1, c2, with_k5)))
        n_act = len(bidx) if has_fc else len(bidx) - 1
        for s in act_shapes[:n_act]:
            scratch.append(pltpu.VMEM(s, jnp.float32))

        if has_fc:
            out_shape = jax.ShapeDtypeStruct((N, 32), jnp.float32)
            out_spec = pl.BlockSpec((_B, 32), lambda n: (n, 0))
        else:
            fs = act_shapes[-1]
            out_shape = jax.ShapeDtypeStruct((N,) + fs[1:], jnp.float32)
            out_spec = pl.BlockSpec(fs, lambda n: (n, 0, 0, 0))

        cur = pl.pallas_call(
            _make_stage_kernel(has_conv0, bidx, has_fc),
            out_shape=out_shape,
            grid=(N // _B,),
            in_specs=in_specs,
            out_specs=out_spec,
            scratch_shapes=scratch,
            compiler_params=pltpu.CompilerParams(
                dimension_semantics=("parallel",),
                vmem_limit_bytes=VMEM_LIMIT),
        )(*operands)
        h = hh

    return cur


# speedup vs baseline: 1.3428x; 1.3428x over previous
import jax
import jax.numpy as jnp
from jax.experimental import pallas as pl
from jax.experimental.pallas import tpu as pltpu

VMEM_LIMIT = 56 * 1024 * 1024

_BLOCKS = [
    (24,  32, False, True),
    (32,  48, False, True),
    (48,  64, True,  True),
    (64,  92, False, True),
    (76, 108, True,  True),
    (76, 108, False, True),
    (76, 108, True,  True),
    (76, 108, False, True),
    (76, 108, True,  False),
]

_STAGES = [
    (True, [0, 1, 2, 3, 4, 5, 6, 7, 8], True),
]

_B = 8


def _names(with_k5):
    return ["s2_2", "s1_0", "s2_0", "s1_2"] + (["s1_1", "s2_1"] if with_k5 else [])


def _cout(c1, c2, with_k5):
    return (3 * c2 + c1) if with_k5 else (2 * c2 + c1)


def _prelu(y, a):
    return jnp.where(y >= 0.0, y, y * a)


def _conv1x1(x2, w, b, a):
    y = jnp.dot(x2, w[...], preferred_element_type=jnp.float32)
    return _prelu(y + b[...], a[...])


def _shift_pool(x4):
    bb, h, w, c = x4.shape
    zc = jnp.zeros((bb, h, 1, c), x4.dtype)
    xw = jnp.concatenate([x4, zc], axis=2)
    zr = jnp.zeros((bb, 1, w + 1, c), x4.dtype)
    xp = jnp.concatenate([xw, zr], axis=1)
    return (xp[:, :h, :w] + xp[:, 1:, :w] + xp[:, :h, 1:] + xp[:, 1:, 1:]) * 0.25


def _pool2(x4):
    bb, h, w, c = x4.shape
    r = x4.reshape(bb, h // 2, 2, w, c)
    rs = r[:, :, 0] + r[:, :, 1]
    s = rs.reshape(bb, h // 2, w // 2, 2, c)
    return (s[:, :, :, 0] + s[:, :, :, 1]) * 0.25


def _convkxk(x2, k, h, w, c1, wref, bref, aref, pad_ref):
    p = (k - 1) // 2
    bb = pad_ref.shape[0]
    hp, wp = h + 2 * p, w + 2 * p
    pad_ref[:, 0:p, :, :] = jnp.zeros((bb, p, wp, c1), jnp.float32)
    pad_ref[:, h + p:hp, :, :] = jnp.zeros((bb, p, wp, c1), jnp.float32)
    pad_ref[:, p:p + h, 0:p, :] = jnp.zeros((bb, h, p, c1), jnp.float32)
    pad_ref[:, p:p + h, w + p:wp, :] = jnp.zeros((bb, h, p, c1), jnp.float32)
    pad_ref[:, p:p + h, p:p + w, :] = x2.reshape(bb, h, w, c1)
    acc = None
    for dy in range(k):
        cols = [pad_ref[:, dy:dy + h, dx:dx + w, :].reshape(bb * h * w, c1)
                for dx in range(k)]
        im = jnp.concatenate(cols, axis=1)
        part = jnp.dot(im, wref[dy * k * c1:(dy + 1) * k * c1, :],
                       preferred_element_type=jnp.float32)
        acc = part if acc is None else acc + part
    return _prelu(acc + bref[...], aref[...])


def _make_stage_kernel(has_conv0, bidx, has_fc):
    cfgs = [_BLOCKS[i] for i in bidx]

    def body(*refs):
        it = iter(refs)
        x_ref = next(it)
        if has_conv0:
            cw, cb, ca = next(it), next(it), next(it)
        prms = []
        for (c1, c2, pool_in, with_k5) in cfgs:
            prms.append({nm: (next(it), next(it), next(it))
                         for nm in _names(with_k5)})
        if has_fc:
            fw, fb = next(it), next(it)
        o_ref = next(it)
        if has_conv0:
            pad0_ref = next(it)
        pads = []
        for (c1, c2, pool_in, with_k5) in cfgs:
            p5 = next(it) if with_k5 else None
            pads.append((p5, next(it)))
        n_act = len(cfgs) if has_fc else len(cfgs) - 1
        acts = [next(it) for _ in range(n_act)]
        bb = _B

        if has_conv0:
            h = w = 16
            y = _convkxk(x_ref[...].reshape(bb * h * w, 16), 3, h, w, 16,
                         cw, cb, ca, pad0_ref)
            prev = (0.25 * (y[:, 0:64] + y[:, 64:128] +
                            y[:, 128:192] + y[:, 192:256])
                    ).reshape(bb, h, w, 64)
        else:
            prev = x_ref[...]
            h = w = prev.shape[1]

        for bi, ((c1, c2, pool_in, with_k5), prm) in enumerate(zip(cfgs, prms)):
            if pool_in:
                prev = _pool2(prev)
                h //= 2
                w //= 2
            cin = prev.shape[-1]
            x2 = prev.reshape(bb * h * w, cin)
            p5_ref, p3_ref = pads[bi]
            out_ref = acts[bi] if bi < len(acts) else o_ref

            s22 = _conv1x1(x2, *prm["s2_2"])
            out_ref[:, :, :, 0:c2] = s22.reshape(bb, h, w, c2)
            off = c2
            if with_k5:
                s11 = _conv1x1(x2, *prm["s1_1"])
                y21 = _convkxk(s11, 5, h, w, c1, *prm["s2_1"], p5_ref)
                out_ref[:, :, :, off:off + c2] = y21.reshape(bb, h, w, c2)
                off += c2
            s10 = _conv1x1(x2, *prm["s1_0"])
            y20 = _convkxk(s10, 3, h, w, c1, *prm["s2_0"], p3_ref)
            out_ref[:, :, :, off:off + c2] = y20.reshape(bb, h, w, c2)
            off += c2
            s12 = _conv1x1(x2, *prm["s1_2"])
            out_ref[:, :, :, off:off + c1] = _shift_pool(s12.reshape(bb, h, w, c1))

            prev = out_ref[...]

        if has_fc:
            flat = prev.reshape(bb, prev.shape[-1])
            o_ref[...] = (jnp.dot(flat, fw[...],
                                  preferred_element_type=jnp.float32) + fb[...])

    return body


def _zero_map(rank):
    def index_map(n):
        return (0,) * rank
    return index_map


def _conv0_s2d_weight(w):
    w2 = jnp.zeros((3, 3, 16, 256), jnp.float32)
    for ty in range(2):
        for tx in range(2):
            for dy5 in range(5):
                for dx5 in range(5):
                    by, sy = divmod(ty - 2 + dy5, 2)
                    bx, sx = divmod(tx - 2 + dx5, 2)
                    ci = (sy * 2 + sx) * 4
                    co = (ty * 2 + tx) * 64
                    w2 = w2.at[by + 1, bx + 1, ci:ci + 4,
                               co:co + 64].set(w[dy5, dx5])
    return w2.reshape(9 * 16, 256)


def kernel(x, conv0_w, conv0_b, conv0_a, *rest):
    N = x.shape[0]
    fc0_w, fc0_b = rest[-2], rest[-1]
    rest = rest[:-2]

    blk_prm = []
    idx = 0
    for (c1, c2, pool_in, with_k5) in _BLOCKS:
        n_sub = 6 if with_k5 else 4
        sub = rest[idx:idx + 3 * n_sub]
        idx += 3 * n_sub
        w22, b22, a22, w10, b10, a10, w20, b20, a20, w12, b12, a12 = sub[:12]
        prm = {"s2_2": (w22, b22, a22), "s1_0": (w10, b10, a10),
               "s2_0": (w20.reshape(9 * c1, c2), b20, a20),
               "s1_2": (w12, b12, a12)}
        if with_k5:
            w11, b11, a11, w21, b21, a21 = sub[12:]
            prm["s1_1"] = (w11, b11, a11)
            prm["s2_1"] = (w21.reshape(25 * c1, c2), b21, a21)
        blk_prm.append(prm)

    cur = x.reshape(N, 4, 16, 2, 16, 2).transpose(0, 2, 4, 3, 5, 1)
    cur = cur.reshape(N, 16, 16, 16)
    cw2 = _conv0_s2d_weight(conv0_w)
    cb2 = jnp.tile(conv0_b, (1, 4))
    ca2 = jnp.tile(conv0_a, (1, 4))

    h = 16
    for (has_conv0, bidx, has_fc) in _STAGES:
        operands = [cur]
        in_specs = [pl.BlockSpec((_B,) + cur.shape[1:],
                                 lambda n: (n, 0, 0, 0))]

        def add(arr):
            operands.append(arr)
            in_specs.append(pl.BlockSpec(arr.shape, _zero_map(arr.ndim)))

        if has_conv0:
            add(cw2)
            add(cb2)
            add(ca2)
        for i in bidx:
            c1, c2, pool_in, with_k5 = _BLOCKS[i]
            for nm in _names(with_k5):
                w, b, a = blk_prm[i][nm]
                add(w); add(b); add(a)
        if has_fc:
            add(fc0_w)
            add(fc0_b)

        scratch = []
        if has_conv0:
            scratch.append(pltpu.VMEM((_B, 18, 18, 16), jnp.float32))
        hh = h
        for i in bidx:
            c1, c2, pool_in, with_k5 = _BLOCKS[i]
            if pool_in:
                hh //= 2
            if with_k5:
                scratch.append(pltpu.VMEM((_B, hh + 4, hh + 4, c1), jnp.float32))
            scratch.append(pltpu.VMEM((_B, hh + 2, hh + 2, c1), jnp.float32))
        hh = h
        act_shapes = []
        for i in bidx:
            c1, c2, pool_in, with_k5 = _BLOCKS[i]
            if pool_in:
                hh //= 2
            act_shapes.append((_B, hh, hh, _cout(c1, c2, with_k5)))
        n_act = len(bidx) if has_fc else len(bidx) - 1
        for s in act_shapes[:n_act]:
            scratch.append(pltpu.VMEM(s, jnp.float32))

        if has_fc:
            out_shape = jax.ShapeDtypeStruct((N, 32), jnp.float32)
            out_spec = pl.BlockSpec((_B, 32), lambda n: (n, 0))
        else:
            fs = act_shapes[-1]
            out_shape = jax.ShapeDtypeStruct((N,) + fs[1:], jnp.float32)
            out_spec = pl.BlockSpec(fs, lambda n: (n, 0, 0, 0))

        cur = pl.pallas_call(
            _make_stage_kernel(has_conv0, bidx, has_fc),
            out_shape=out_shape,
            grid=(N // _B,),
            in_specs=in_specs,
            out_specs=out_spec,
            scratch_shapes=scratch,
            compiler_params=pltpu.CompilerParams(
                dimension_semantics=("parallel",),
                vmem_limit_bytes=VMEM_LIMIT),
        )(*operands)
        h = hh

    return cur
